# baseline (speedup 1.0000x reference)
import functools

import jax
import jax.numpy as jnp
from jax import lax
from jax.experimental import pallas as pl
from jax.experimental.pallas import tpu as pltpu
from jax.experimental.pallas import tpu_sc as plsc

NC = 2
NS = 16
NW = NC * NS

C = 128
NBUF = 5


def _make_sc_gather(B, D, n_chunks):
    assert n_chunks % NBUF == 0
    n_groups = n_chunks // NBUF
    b_per_w = n_chunks * C
    mesh = plsc.VectorSubcoreMesh(core_axis_name="c", subcore_axis_name="s")

    scratch = [pltpu.VMEM((b_per_w,), jnp.int32)]
    scratch += [pltpu.VMEM((C, D), jnp.float32) for _ in range(NBUF)]
    scratch += [pltpu.SemaphoreType.DMA for _ in range(2 * NBUF)]

    @functools.partial(
        pl.kernel,
        mesh=mesh,
        out_type=jax.ShapeDtypeStruct((B, D), jnp.float32),
        scratch_types=scratch,
    )
    def k(table_hbm, idx_hbm, out_hbm, idx_v, *rest):
        rows = rest[:NBUF]
        gsem = rest[NBUF:2 * NBUF]
        wsem = rest[2 * NBUF:]
        wid = lax.axis_index("s") * NC + lax.axis_index("c")
        base = wid * b_per_w
        pltpu.sync_copy(
            idx_hbm.at[pl.ds(pl.multiple_of(base, 8), b_per_w)], idx_v)

        def idx_slice(i):
            return idx_v.at[pl.ds(pl.multiple_of(i * C, 8), C)]

        def start_gather(i, b):
            pltpu.async_copy(table_hbm.at[idx_slice(i)], rows[b], gsem[b])

        def wait_gather(b):
            pltpu.make_async_copy(
                table_hbm.at[idx_slice(0)], rows[b], gsem[b]).wait()

        def start_write(i, b):
            pltpu.async_copy(
                rows[b], out_hbm.at[pl.ds(base + i * C, C)], wsem[b])

        def wait_write(b):
            pltpu.make_async_copy(
                rows[b], out_hbm.at[pl.ds(base, C)], wsem[b]).wait()

        for b in range(NBUF):
            start_gather(b, b)
        for b in range(NBUF):
            wait_gather(b)
            start_write(b, b)

        def body(j, _):
            i0 = (j + 1) * NBUF
            for b in range(NBUF):
                wait_write(b)
                start_gather(i0 + b, b)
            for b in range(NBUF):
                wait_gather(b)
                start_write(i0 + b, b)
            return 0

        lax.fori_loop(0, n_groups - 1, body, 0)
        for b in range(NBUF):
            wait_write(b)

    return k


def kernel(seq, table):
    S0, T = seq.shape
    B = S0 * T
    D = table.shape[1]
    assert B % (NW * C) == 0
    n_chunks = B // (NW * C)
    idx = seq.T.astype(jnp.int32).reshape(B)
    out = _make_sc_gather(B, D, n_chunks)(table, idx)
    return out.reshape(T, S0, D).transpose(1, 0, 2)


# speedup vs baseline: 1.0182x; 1.0182x over previous
import functools

import jax
import jax.numpy as jnp
from jax import lax
from jax.experimental import pallas as pl
from jax.experimental.pallas import tpu as pltpu
from jax.experimental.pallas import tpu_sc as plsc

NC = 2
NS = 16
NW = NC * NS

C = 64
NBUF = 10


def _make_sc_gather(B, D, n_chunks):
    assert n_chunks % NBUF == 0
    n_groups = n_chunks // NBUF
    b_per_w = n_chunks * C
    mesh = plsc.VectorSubcoreMesh(core_axis_name="c", subcore_axis_name="s")

    npair = NBUF // 2
    scratch = [pltpu.VMEM((b_per_w,), jnp.int32)]
    scratch += [pltpu.VMEM((NBUF * C, D), jnp.float32)]
    scratch += [pltpu.SemaphoreType.DMA for _ in range(NBUF + npair)]

    @functools.partial(
        pl.kernel,
        mesh=mesh,
        out_type=jax.ShapeDtypeStruct((B, D), jnp.float32),
        scratch_types=scratch,
    )
    def k(table_hbm, idx_hbm, out_hbm, idx_v, rows, *sems):
        gsem = sems[:NBUF]
        wsem = sems[NBUF:]
        wid = lax.axis_index("s") * NC + lax.axis_index("c")
        base = wid * b_per_w
        pltpu.sync_copy(
            idx_hbm.at[pl.ds(pl.multiple_of(base, 8), b_per_w)], idx_v)

        def idx_slice(i):
            return idx_v.at[pl.ds(pl.multiple_of(i * C, 8), C)]

        def start_gather(i, b):
            pltpu.async_copy(
                table_hbm.at[idx_slice(i)],
                rows.at[pl.ds(b * C, C)], gsem[b])

        def wait_gather(b):
            pltpu.make_async_copy(
                table_hbm.at[idx_slice(0)],
                rows.at[pl.ds(b * C, C)], gsem[b]).wait()

        def start_write(i, p):
            pltpu.async_copy(
                rows.at[pl.ds(2 * p * C, 2 * C)],
                out_hbm.at[pl.ds(base + i * C, 2 * C)], wsem[p])

        def wait_write(p):
            pltpu.make_async_copy(
                rows.at[pl.ds(2 * p * C, 2 * C)],
                out_hbm.at[pl.ds(base, 2 * C)], wsem[p]).wait()

        for b in range(NBUF):
            start_gather(b, b)
        for p in range(npair):
            wait_gather(2 * p)
            wait_gather(2 * p + 1)
            start_write(2 * p, p)

        def body(j, _):
            i0 = (j + 1) * NBUF
            for b in range(NBUF):
                if b % 2 == 0:
                    wait_write(b // 2)
                start_gather(i0 + b, b)
            for p in range(npair):
                wait_gather(2 * p)
                wait_gather(2 * p + 1)
                start_write(i0 + 2 * p, p)
            return 0

        lax.fori_loop(0, n_groups - 1, body, 0)
        for p in range(npair):
            wait_write(p)

    return k


def kernel(seq, table):
    S0, T = seq.shape
    B = S0 * T
    D = table.shape[1]
    assert B % (NW * C) == 0
    n_chunks = B // (NW * C)
    idx = seq.T.astype(jnp.int32).reshape(B)
    out = _make_sc_gather(B, D, n_chunks)(table, idx)
    return out.reshape(T, S0, D).transpose(1, 0, 2)


# speedup vs baseline: 1.0265x; 1.0081x over previous
import functools

import jax
import jax.numpy as jnp
from jax import lax
from jax.experimental import pallas as pl
from jax.experimental.pallas import tpu as pltpu
from jax.experimental.pallas import tpu_sc as plsc

NC = 2
NS = 16
NW = NC * NS

C = 64
NBUF = 10
WGRP = 5


def _make_sc_gather(B, D, n_chunks):
    assert n_chunks % NBUF == 0
    n_groups = n_chunks // NBUF
    b_per_w = n_chunks * C
    mesh = plsc.VectorSubcoreMesh(core_axis_name="c", subcore_axis_name="s")

    nwg = NBUF // WGRP
    scratch = [pltpu.VMEM((b_per_w,), jnp.int32)]
    scratch += [pltpu.VMEM((NBUF * C, D), jnp.float32)]
    scratch += [pltpu.SemaphoreType.DMA for _ in range(NBUF + nwg)]

    @functools.partial(
        pl.kernel,
        mesh=mesh,
        out_type=jax.ShapeDtypeStruct((B, D), jnp.float32),
        scratch_types=scratch,
    )
    def k(table_hbm, idx_hbm, out_hbm, idx_v, rows, *sems):
        gsem = sems[:NBUF]
        wsem = sems[NBUF:]
        wid = lax.axis_index("s") * NC + lax.axis_index("c")
        base = wid * b_per_w
        pltpu.sync_copy(
            idx_hbm.at[pl.ds(pl.multiple_of(base, 8), b_per_w)], idx_v)

        def idx_slice(i):
            return idx_v.at[pl.ds(pl.multiple_of(i * C, 8), C)]

        def start_gather(i, b):
            pltpu.async_copy(
                table_hbm.at[idx_slice(i)],
                rows.at[pl.ds(b * C, C)], gsem[b])

        def wait_gather(b):
            pltpu.make_async_copy(
                table_hbm.at[idx_slice(0)],
                rows.at[pl.ds(b * C, C)], gsem[b]).wait()

        def start_write(i, p):
            pltpu.async_copy(
                rows.at[pl.ds(WGRP * p * C, WGRP * C)],
                out_hbm.at[pl.ds(base + i * C, WGRP * C)], wsem[p])

        def wait_write(p):
            pltpu.make_async_copy(
                rows.at[pl.ds(WGRP * p * C, WGRP * C)],
                out_hbm.at[pl.ds(base, WGRP * C)], wsem[p]).wait()

        for b in range(NBUF):
            start_gather(b, b)
        for p in range(nwg):
            for q in range(WGRP):
                wait_gather(WGRP * p + q)
            start_write(WGRP * p, p)

        def body(j, _):
            i0 = (j + 1) * NBUF
            for b in range(NBUF):
                if b % WGRP == 0:
                    wait_write(b // WGRP)
                start_gather(i0 + b, b)
            for p in range(nwg):
                for q in range(WGRP):
                    wait_gather(WGRP * p + q)
                start_write(i0 + WGRP * p, p)
            return 0

        lax.fori_loop(0, n_groups - 1, body, 0)
        for p in range(nwg):
            wait_write(p)

    return k


def kernel(seq, table):
    S0, T = seq.shape
    B = S0 * T
    D = table.shape[1]
    assert B % (NW * C) == 0
    n_chunks = B // (NW * C)
    idx = seq.T.astype(jnp.int32).reshape(B)
    out = _make_sc_gather(B, D, n_chunks)(table, idx)
    return out.reshape(T, S0, D).transpose(1, 0, 2)
